# Initial kernel scaffold; baseline (speedup 1.0000x reference)
import functools

import jax
import jax.numpy as jnp
from jax import lax
from jax.experimental import pallas as pl
from jax.experimental.pallas import tpu as pltpu
from jax.experimental.pallas import tpu_sc as plsc

_ND = 10000
_H = 128
_HW = 144
_NPX = 10240
_NU = 10016
_E = 320000
_EL = _E + _ND
_TILE_E = 10368
_EPAD = 32 * _TILE_E
_NB = 81
_EB = 128
_NC = 2
_NS = 16
_ZROWS = _NU // _NS
_B = 1024
_IDS = 2 * _B

_HIGH = lax.Precision.HIGHEST


def _proj_body(x_ref, ws_ref, wd_ref, as_ref, ad_ref, hs_ref, sc_ref):
    x = x_ref[...]
    hs = jnp.dot(x, ws_ref[...], preferred_element_type=jnp.float32,
                 precision=_HIGH)
    ss = lax.dot_general(as_ref[...], hs, (((1,), (1,)), ((), ())),
                         preferred_element_type=jnp.float32, precision=_HIGH)
    wdv = lax.dot_general(ad_ref[...], wd_ref[...], (((1,), (1,)), ((), ())),
                          preferred_element_type=jnp.float32, precision=_HIGH)
    sd = lax.dot_general(wdv, x, (((1,), (1,)), ((), ())),
                         preferred_element_type=jnp.float32, precision=_HIGH)
    hs_ref[:, 0:_H] = hs
    col = lax.broadcasted_iota(jnp.int32, (x.shape[0], _HW - _H), 1)
    hs_ref[:, _H:_HW] = jnp.where(col == 0, 1.0, 0.0).astype(jnp.float32)
    sc_ref[0:1, :] = ss
    sc_ref[1:2, :] = sd
    sc_ref[2:8, :] = jnp.zeros((6, ss.shape[1]), jnp.float32)


def _proj(x_pad, Ws, Wd, a_s, a_d):
    blk = 1280
    grid = _NPX // blk
    return pl.pallas_call(
        _proj_body,
        grid=(grid,),
        in_specs=[
            pl.BlockSpec((blk, _H), lambda g: (g, 0)),
            pl.BlockSpec((_H, _H), lambda g: (0, 0)),
            pl.BlockSpec((_H, _H), lambda g: (0, 0)),
            pl.BlockSpec((1, _H), lambda g: (0, 0)),
            pl.BlockSpec((1, _H), lambda g: (0, 0)),
        ],
        out_specs=[
            pl.BlockSpec((blk, _HW), lambda g: (g, 0)),
            pl.BlockSpec((8, blk), lambda g: (0, g)),
        ],
        out_shape=[
            jax.ShapeDtypeStruct((_NPX, _HW), jnp.float32),
            jax.ShapeDtypeStruct((8, _NPX), jnp.float32),
        ],
    )(x_pad, Ws, Wd, a_s, a_d)


def _sc_gat_body(src_hbm, dst_hbm, scores_hbm, hsaug_hbm, ids_hbm,
                 pout_hbm, mout_hbm,
                 src_v, dst_v, e_v, ss_v, sd_v, rows_v, mbuf_v, mall_v,
                 ids_v, u_sh, msh_sh, gsem):
    cid = lax.axis_index("c")
    sid = lax.axis_index("s")
    chunk = cid * _NS + sid

    pltpu.sync_copy(src_hbm.at[pl.ds(chunk * _NB, _NB)], src_v)
    pltpu.sync_copy(dst_hbm.at[pl.ds(chunk * _NB, _NB)], dst_v)
    pltpu.sync_copy(scores_hbm.at[0], ss_v)
    pltpu.sync_copy(scores_hbm.at[1], sd_v)

    def passa(b, mx):
        for c in range(8):
            off = c * 16
            isrc = src_v[b, pl.ds(off, 16)]
            idst = dst_v[b, pl.ds(off, 16)]
            a = plsc.load_gather(ss_v, [isrc])
            d = plsc.load_gather(sd_v, [idst])
            e = a + d
            e = jnp.where(e >= 0.0, e, 0.2 * e)
            e_v[b, pl.ds(off, 16)] = e
            mx = jnp.maximum(mx, e)
        return mx
    mx = lax.fori_loop(0, _NB, passa,
                       jnp.full((16,), -jnp.inf, jnp.float32))

    mbuf_v[...] = mx
    pltpu.sync_copy(mbuf_v, msh_sh.at[sid])
    plsc.subcore_barrier()
    pltpu.sync_copy(msh_sh, mall_v)
    for t in range(_NS):
        mx = jnp.maximum(mx, mall_v[t, :])
    m_core = jnp.max(mx)

    @pl.when(sid == 0)
    def _():
        mbuf_v[...] = jnp.full((16,), m_core, jnp.float32)
        pltpu.sync_copy(mbuf_v, mout_hbm.at[pl.ds(cid * 16, 16)])

    def passb(b, carry):
        for c in range(8):
            off = c * 16
            e = e_v[b, pl.ds(off, 16)]
            e_v[b, pl.ds(off, 16)] = jnp.exp(e - m_core)
        return carry
    lax.fori_loop(0, _NB, passb, 0)

    zv = jnp.zeros((16,), jnp.float32)
    def zrow(r, carry):
        for c in range(_HW // 16):
            rows_v[r, pl.ds(c * 16, 16)] = zv
        return carry
    lax.fori_loop(0, _EB, zrow, 0)
    for k in range(4):
        pltpu.sync_copy(rows_v, u_sh.at[pl.ds(sid * _ZROWS + k * _EB, _EB)])
    pltpu.sync_copy(rows_v.at[pl.ds(0, _ZROWS - 4 * _EB)],
                    u_sh.at[pl.ds(sid * _ZROWS + 4 * _EB, _ZROWS - 4 * _EB)])
    plsc.subcore_barrier()

    def passc(g, carry):
        pltpu.async_copy(hsaug_hbm.at[src_v.at[g]], rows_v, gsem).wait()
        def scale(r, c2):
            ex = e_v[g, r]
            for c in range(_HW // 16):
                xv = rows_v[r, pl.ds(c * 16, 16)]
                rows_v[r, pl.ds(c * 16, 16)] = xv * ex
            return c2
        lax.fori_loop(0, _EB, scale, 0)
        pltpu.sync_copy(rows_v, u_sh.at[dst_v.at[g]], add=True)
        return carry
    lax.fori_loop(0, _NB, passc, 0)
    plsc.subcore_barrier()

    pltpu.sync_copy(ids_hbm.at[sid], ids_v)
    pltpu.async_copy(u_sh.at[ids_v], rows_v, gsem).wait()
    pltpu.sync_copy(rows_v,
                    pout_hbm.at[pl.ds(cid * _IDS + sid * _EB, _EB)])


def _sc_gat(src2, dst2, scores, hsaug, ids2):
    mesh = plsc.VectorSubcoreMesh(core_axis_name="c", subcore_axis_name="s",
                                  num_cores=_NC, num_subcores=_NS)
    f = pl.kernel(
        _sc_gat_body,
        out_type=[
            jax.ShapeDtypeStruct((_NC * _IDS, _HW), jnp.float32),
            jax.ShapeDtypeStruct((_NC * 16,), jnp.float32),
        ],
        mesh=mesh,
        scratch_types=[
            pltpu.VMEM((_NB, _EB), jnp.int32),
            pltpu.VMEM((_NB, _EB), jnp.int32),
            pltpu.VMEM((_NB, _EB), jnp.float32),
            pltpu.VMEM((_NPX,), jnp.float32),
            pltpu.VMEM((_NPX,), jnp.float32),
            pltpu.VMEM((_EB, _HW), jnp.float32),
            pltpu.VMEM((16,), jnp.float32),
            pltpu.VMEM((_NS, 16), jnp.float32),
            pltpu.VMEM((_EB,), jnp.int32),
            pltpu.VMEM_SHARED((_NU, _HW), jnp.float32),
            pltpu.VMEM_SHARED((_NS, 16), jnp.float32),
            pltpu.SemaphoreType.DMA,
        ],
    )
    return f(src2, dst2, scores, hsaug, ids2)


def _cell_body(cf_ref, r1_ref, b1_ref, r2_ref, b2_ref, r3_ref, b3_ref,
               out_ref):
    cf = cf_ref[...]
    n = jnp.sqrt(jnp.sum(cf * cf, axis=1, keepdims=True))
    cf = cf / jnp.maximum(n, 1e-12)
    h = jnp.maximum(jnp.dot(cf, r1_ref[...], preferred_element_type=jnp.float32,
                            precision=_HIGH) + b1_ref[...], 0.0)
    h = jnp.maximum(jnp.dot(h, r2_ref[...], preferred_element_type=jnp.float32,
                            precision=_HIGH) + b2_ref[...], 0.0)
    h = jnp.maximum(jnp.dot(h, r3_ref[...], preferred_element_type=jnp.float32,
                            precision=_HIGH) + b3_ref[...], 0.0)
    out_ref[...] = h


def _cell_mlp(cf, R1, rb1, R2, rb2, R3, rb3):
    return pl.pallas_call(
        _cell_body,
        out_shape=jax.ShapeDtypeStruct((_B, 2 * _H), jnp.float32),
    )(cf, R1, rb1.reshape(1, -1), R2, rb2.reshape(1, -1), R3,
      rb3.reshape(1, -1))


def _final_body(p_ref, m_ref, bdd_ref, cell_ref, s1_ref, sb1_ref, s2_ref,
                sb2_ref, s3_ref, sb3_ref, c_ref, cb_ref, out_ref):
    m = m_ref[...]
    m0 = m[0:1, 0:1]
    m1 = m[0:1, 16:17]
    mg = jnp.maximum(m0, m1)
    f0 = jnp.exp(m0 - mg)
    f1 = jnp.exp(m1 - mg)
    p = p_ref[...]
    comb = p[0:_IDS] * f0 + p[_IDS:2 * _IDS] * f1
    den = comb[:, _H:_H + 1]
    d = comb[:, 0:_H] / (den + 1e-16) + bdd_ref[...]
    d = jnp.maximum(d, 0.0)
    hidden = jnp.concatenate([d[0:_B], d[_B:2 * _B], cell_ref[...]], axis=1)
    n = jnp.sqrt(jnp.sum(hidden * hidden, axis=1, keepdims=True))
    hidden = hidden / jnp.maximum(n, 1e-12)
    h = jnp.maximum(jnp.dot(hidden, s1_ref[...],
                            preferred_element_type=jnp.float32,
                            precision=_HIGH) + sb1_ref[...], 0.0)
    h = jnp.maximum(jnp.dot(h, s2_ref[...], preferred_element_type=jnp.float32,
                            precision=_HIGH) + sb2_ref[...], 0.0)
    h = jnp.maximum(jnp.dot(h, s3_ref[...], preferred_element_type=jnp.float32,
                            precision=_HIGH) + sb3_ref[...], 0.0)
    out_ref[...] = jnp.dot(h, c_ref[...], preferred_element_type=jnp.float32,
                           precision=_HIGH) + cb_ref[...]


def _final_mlp(pout, mout, bdd, cell, S1, sb1, S2, sb2, S3, sb3, C, cb):
    return pl.pallas_call(
        _final_body,
        out_shape=jax.ShapeDtypeStruct((_B, 2), jnp.float32),
    )(pout, mout.reshape(1, -1), bdd.reshape(1, -1), cell, S1,
      sb1.reshape(1, -1), S2, sb2.reshape(1, -1), S3, sb3.reshape(1, -1),
      C, cb.reshape(1, -1))


def kernel(x_drug, x_target, cell_features, edge_index_dd, edge_index_dt,
           edge_index_tt, drug1_id, drug2_id, Wdd_s, Wdd_d, add_s, add_d,
           bdd, Wdt_s, Wdt_d, adt_s, adt_d, bdt, Wtt_s, Wtt_d, att_s, att_d,
           btt, R1, rb1, R2, rb2, R3, rb3, S1, sb1, S2, sb2, S3, sb3, C, cb):
    ei = edge_index_dd.astype(jnp.int32)
    loop = jnp.arange(_ND, dtype=jnp.int32)
    src = jnp.concatenate(
        [ei[0], loop, jnp.zeros((_EPAD - _EL,), jnp.int32)])
    dst = jnp.concatenate(
        [ei[1], loop, jnp.full((_EPAD - _EL,), _ND, jnp.int32)])
    src2 = src.reshape(_EPAD // _EB, _EB)
    dst2 = dst.reshape(_EPAD // _EB, _EB)
    ids2 = jnp.concatenate([drug1_id.astype(jnp.int32),
                            drug2_id.astype(jnp.int32)]).reshape(_NS, _EB)
    x_pad = jnp.pad(x_drug, ((0, _NPX - _ND), (0, 0)))

    cell = _cell_mlp(cell_features, R1, rb1, R2, rb2, R3, rb3)
    hsaug, scores = _proj(x_pad, Wdd_s, Wdd_d, add_s.reshape(1, _H),
                          add_d.reshape(1, _H))
    pout, mout = _sc_gat(src2, dst2, scores, hsaug, ids2)
    return _final_mlp(pout, mout, bdd, cell, S1, sb1, S2, sb2, S3, sb3, C, cb)


# speedup vs baseline: 6.7675x; 6.7675x over previous
import functools

import jax
import jax.numpy as jnp
from jax import lax
from jax.experimental import pallas as pl
from jax.experimental.pallas import tpu as pltpu
from jax.experimental.pallas import tpu_sc as plsc

_ND = 10000
_H = 128
_HW = 144
_NPX = 10240
_NU = 10112
_E = 320000
_EL = _E + _ND
_TILE_E = 11264
_EPAD = 32 * _TILE_E
_EROW = 64
_ROWS_T = _TILE_E // _EROW
_CH = 16
_NCH = _ROWS_T // _CH
_NC = 2
_NS = 16
_ZROWS = _NU // _NS
_B = 1024
_IDS = 2 * _B

_HIGH = lax.Precision.HIGHEST


def _proj_body(x_ref, ws_ref, wd_ref, as_ref, ad_ref, hs_ref, sc_ref):
    x = x_ref[...]
    hs = jnp.dot(x, ws_ref[...], preferred_element_type=jnp.float32,
                 precision=_HIGH)
    ss = lax.dot_general(as_ref[...], hs, (((1,), (1,)), ((), ())),
                         preferred_element_type=jnp.float32, precision=_HIGH)
    wdv = lax.dot_general(ad_ref[...], wd_ref[...], (((1,), (1,)), ((), ())),
                          preferred_element_type=jnp.float32, precision=_HIGH)
    sd = lax.dot_general(wdv, x, (((1,), (1,)), ((), ())),
                         preferred_element_type=jnp.float32, precision=_HIGH)
    hs_ref[:, 0:_H] = hs
    col = lax.broadcasted_iota(jnp.int32, (x.shape[0], _HW - _H), 1)
    hs_ref[:, _H:_HW] = jnp.where(col == 0, 1.0, 0.0).astype(jnp.float32)
    sc_ref[0:1, :] = ss
    sc_ref[1:2, :] = sd
    sc_ref[2:8, :] = jnp.zeros((6, ss.shape[1]), jnp.float32)


def _proj(x_pad, Ws, Wd, a_s, a_d):
    blk = 1280
    grid = _NPX // blk
    return pl.pallas_call(
        _proj_body,
        grid=(grid,),
        in_specs=[
            pl.BlockSpec((blk, _H), lambda g: (g, 0)),
            pl.BlockSpec((_H, _H), lambda g: (0, 0)),
            pl.BlockSpec((_H, _H), lambda g: (0, 0)),
            pl.BlockSpec((1, _H), lambda g: (0, 0)),
            pl.BlockSpec((1, _H), lambda g: (0, 0)),
        ],
        out_specs=[
            pl.BlockSpec((blk, _HW), lambda g: (g, 0)),
            pl.BlockSpec((8, blk), lambda g: (0, g)),
        ],
        out_shape=[
            jax.ShapeDtypeStruct((_NPX, _HW), jnp.float32),
            jax.ShapeDtypeStruct((8, _NPX), jnp.float32),
        ],
    )(x_pad, Ws, Wd, a_s, a_d)


def _sc_gat_body(src_hbm, dst_hbm, ss_hbm, sd_hbm, hsaug_hbm, ids_hbm,
                 pout_hbm, mout_hbm,
                 src_c, dst_c, ex_row, ss_v, sd_v, rows_v, mbuf_v, mall_v,
                 ids_v, u_sh, msh_sh, gsem):
    cid = lax.axis_index("c")
    sid = lax.axis_index("s")
    tile = cid * _NS + sid
    base = tile * _ROWS_T

    pltpu.sync_copy(ss_hbm, ss_v)
    pltpu.sync_copy(sd_hbm, sd_v)

    def _logit(r, c):
        off = c * 16
        isrc = src_c[r, pl.ds(off, 16)]
        idst = dst_c[r, pl.ds(off, 16)]
        e = plsc.load_gather(ss_v, [isrc]) + plsc.load_gather(sd_v, [idst])
        return jnp.where(e >= 0.0, e, 0.2 * e)

    def passa(k, mx):
        pltpu.sync_copy(src_hbm.at[pl.ds(base + k * _CH, _CH)], src_c)
        pltpu.sync_copy(dst_hbm.at[pl.ds(base + k * _CH, _CH)], dst_c)
        def rowloop(r, mx2):
            for c in range(_EROW // 16):
                mx2 = jnp.maximum(mx2, _logit(r, c))
            return mx2
        return lax.fori_loop(0, _CH, rowloop, mx)
    mx = lax.fori_loop(0, _NCH, passa,
                       jnp.full((16,), -jnp.inf, jnp.float32))

    mbuf_v[...] = mx
    pltpu.sync_copy(mbuf_v, msh_sh.at[pl.ds(sid * 16, 16)])
    plsc.subcore_barrier()
    pltpu.sync_copy(msh_sh, mall_v)
    for t in range(_NS):
        mx = jnp.maximum(mx, mall_v[pl.ds(t * 16, 16)])
    m_core = jnp.max(mx)

    @pl.when(sid == 0)
    def _():
        mbuf_v[...] = jnp.full((16,), m_core, jnp.float32)
        pltpu.sync_copy(mbuf_v, mout_hbm.at[pl.ds(cid * 16, 16)])

    zv = jnp.zeros((16,), jnp.float32)
    def zrow(r, carry):
        for c in range(_HW // 16):
            rows_v[r, pl.ds(c * 16, 16)] = zv
        return carry
    lax.fori_loop(0, _EROW, zrow, 0)
    nfull = _ZROWS // _EROW
    for k in range(nfull):
        pltpu.sync_copy(rows_v, u_sh.at[pl.ds(sid * _ZROWS + k * _EROW,
                                              _EROW)])
    rem = _ZROWS - nfull * _EROW
    if rem:
        pltpu.sync_copy(rows_v.at[pl.ds(0, rem)],
                        u_sh.at[pl.ds(sid * _ZROWS + nfull * _EROW, rem)])
    plsc.subcore_barrier()

    def passc(k, carry):
        pltpu.sync_copy(src_hbm.at[pl.ds(base + k * _CH, _CH)], src_c)
        pltpu.sync_copy(dst_hbm.at[pl.ds(base + k * _CH, _CH)], dst_c)
        def rowloop(r, c2):
            for c in range(_EROW // 16):
                ex_row[pl.ds(c * 16, 16)] = jnp.exp(_logit(r, c) - m_core)
            pltpu.async_copy(hsaug_hbm.at[src_c.at[r]], rows_v, gsem).wait()
            def scale(q, c3):
                ex16 = ex_row[pl.ds(q * 16, 16)]
                for l in range(16):
                    rr = q * 16 + l
                    ex = ex16[l]
                    for c in range(_HW // 16):
                        xv = rows_v[rr, pl.ds(c * 16, 16)]
                        rows_v[rr, pl.ds(c * 16, 16)] = xv * ex
                return c3
            lax.fori_loop(0, _EROW // 16, scale, 0)
            pltpu.sync_copy(rows_v, u_sh.at[dst_c.at[r]], add=True)
            return c2
        return lax.fori_loop(0, _CH, rowloop, carry)
    lax.fori_loop(0, _NCH, passc, 0)
    plsc.subcore_barrier()

    for b in range(2):
        pltpu.sync_copy(ids_hbm.at[pl.ds(sid * 128 + b * _EROW, _EROW)],
                        ids_v)
        pltpu.async_copy(u_sh.at[ids_v], rows_v, gsem).wait()
        pltpu.sync_copy(rows_v.at[pl.ds(0, _EROW)],
                        pout_hbm.at[pl.ds(cid * _IDS + sid * 128 + b * _EROW,
                                          _EROW)])


def _sc_gat(src2, dst2, ss, sd, hsaug, ids1):
    mesh = plsc.VectorSubcoreMesh(core_axis_name="c", subcore_axis_name="s",
                                  num_cores=_NC, num_subcores=_NS)
    f = pl.kernel(
        _sc_gat_body,
        out_type=[
            jax.ShapeDtypeStruct((_NC * _IDS, _HW), jnp.float32),
            jax.ShapeDtypeStruct((_NC * 16,), jnp.float32),
        ],
        mesh=mesh,
        scratch_types=[
            pltpu.VMEM((_CH, _EROW), jnp.int32),
            pltpu.VMEM((_CH, _EROW), jnp.int32),
            pltpu.VMEM((_EROW,), jnp.float32),
            pltpu.VMEM((_NPX,), jnp.float32),
            pltpu.VMEM((_NPX,), jnp.float32),
            pltpu.VMEM((_EROW, _HW), jnp.float32),
            pltpu.VMEM((16,), jnp.float32),
            pltpu.VMEM((_NS * 16,), jnp.float32),
            pltpu.VMEM((_EROW,), jnp.int32),
            pltpu.VMEM_SHARED((_NU, _HW), jnp.float32),
            pltpu.VMEM_SHARED((_NS * 16,), jnp.float32),
            pltpu.SemaphoreType.DMA,
        ],
        compiler_params=pltpu.CompilerParams(use_tc_tiling_on_sc=False,
                                             needs_layout_passes=False),
    )
    return f(src2, dst2, ss, sd, hsaug, ids1)


def _cell_body(cf_ref, r1_ref, b1_ref, r2_ref, b2_ref, r3_ref, b3_ref,
               out_ref):
    cf = cf_ref[...]
    n = jnp.sqrt(jnp.sum(cf * cf, axis=1, keepdims=True))
    cf = cf / jnp.maximum(n, 1e-12)
    h = jnp.maximum(jnp.dot(cf, r1_ref[...], preferred_element_type=jnp.float32,
                            precision=_HIGH) + b1_ref[...], 0.0)
    h = jnp.maximum(jnp.dot(h, r2_ref[...], preferred_element_type=jnp.float32,
                            precision=_HIGH) + b2_ref[...], 0.0)
    h = jnp.maximum(jnp.dot(h, r3_ref[...], preferred_element_type=jnp.float32,
                            precision=_HIGH) + b3_ref[...], 0.0)
    out_ref[...] = h


def _cell_mlp(cf, R1, rb1, R2, rb2, R3, rb3):
    return pl.pallas_call(
        _cell_body,
        out_shape=jax.ShapeDtypeStruct((_B, 2 * _H), jnp.float32),
    )(cf, R1, rb1.reshape(1, -1), R2, rb2.reshape(1, -1), R3,
      rb3.reshape(1, -1))


def _final_body(p_ref, m_ref, bdd_ref, cell_ref, s1_ref, sb1_ref, s2_ref,
                sb2_ref, s3_ref, sb3_ref, c_ref, cb_ref, out_ref):
    m = m_ref[...]
    m0 = m[0:1, 0:1]
    m1 = m[0:1, 16:17]
    mg = jnp.maximum(m0, m1)
    f0 = jnp.exp(m0 - mg)
    f1 = jnp.exp(m1 - mg)
    p = p_ref[...]
    comb = p[0:_IDS] * f0 + p[_IDS:2 * _IDS] * f1
    den = comb[:, _H:_H + 1]
    d = comb[:, 0:_H] / (den + 1e-16) + bdd_ref[...]
    d = jnp.maximum(d, 0.0)
    hidden = jnp.concatenate([d[0:_B], d[_B:2 * _B], cell_ref[...]], axis=1)
    n = jnp.sqrt(jnp.sum(hidden * hidden, axis=1, keepdims=True))
    hidden = hidden / jnp.maximum(n, 1e-12)
    h = jnp.maximum(jnp.dot(hidden, s1_ref[...],
                            preferred_element_type=jnp.float32,
                            precision=_HIGH) + sb1_ref[...], 0.0)
    h = jnp.maximum(jnp.dot(h, s2_ref[...], preferred_element_type=jnp.float32,
                            precision=_HIGH) + sb2_ref[...], 0.0)
    h = jnp.maximum(jnp.dot(h, s3_ref[...], preferred_element_type=jnp.float32,
                            precision=_HIGH) + sb3_ref[...], 0.0)
    out_ref[...] = jnp.dot(h, c_ref[...], preferred_element_type=jnp.float32,
                           precision=_HIGH) + cb_ref[...]


def _final_mlp(pout, mout, bdd, cell, S1, sb1, S2, sb2, S3, sb3, C, cb):
    return pl.pallas_call(
        _final_body,
        out_shape=jax.ShapeDtypeStruct((_B, 2), jnp.float32),
    )(pout, mout.reshape(1, -1), bdd.reshape(1, -1), cell, S1,
      sb1.reshape(1, -1), S2, sb2.reshape(1, -1), S3, sb3.reshape(1, -1),
      C, cb.reshape(1, -1))


def kernel(x_drug, x_target, cell_features, edge_index_dd, edge_index_dt,
           edge_index_tt, drug1_id, drug2_id, Wdd_s, Wdd_d, add_s, add_d,
           bdd, Wdt_s, Wdt_d, adt_s, adt_d, bdt, Wtt_s, Wtt_d, att_s, att_d,
           btt, R1, rb1, R2, rb2, R3, rb3, S1, sb1, S2, sb2, S3, sb3, C, cb):
    ei = edge_index_dd.astype(jnp.int32)
    loop = jnp.arange(_ND, dtype=jnp.int32)
    src = jnp.concatenate(
        [ei[0], loop, jnp.zeros((_EPAD - _EL,), jnp.int32)])
    dst = jnp.concatenate(
        [ei[1], loop, jnp.full((_EPAD - _EL,), _ND, jnp.int32)])
    src2 = src.reshape(_EPAD // _EROW, _EROW)
    dst2 = dst.reshape(_EPAD // _EROW, _EROW)
    ids1 = jnp.concatenate([drug1_id.astype(jnp.int32),
                            drug2_id.astype(jnp.int32)])
    x_pad = jnp.pad(x_drug, ((0, _NPX - _ND), (0, 0)))

    cell = _cell_mlp(cell_features, R1, rb1, R2, rb2, R3, rb3)
    hsaug, scores = _proj(x_pad, Wdd_s, Wdd_d, add_s.reshape(1, _H),
                          add_d.reshape(1, _H))
    pout, mout = _sc_gat(src2, dst2, scores[0], scores[1], hsaug, ids1)
    return _final_mlp(pout, mout, bdd, cell, S1, sb1, S2, sb2, S3, sb3, C, cb)


# speedup vs baseline: 7.5551x; 1.1164x over previous
import functools

import jax
import jax.numpy as jnp
from jax import lax
from jax.experimental import pallas as pl
from jax.experimental.pallas import tpu as pltpu
from jax.experimental.pallas import tpu_sc as plsc

_ND = 10000
_H = 128
_HW = 144
_NPX = 10240
_NU = 10112
_E = 320000
_EL = _E + _ND
_TILE_E = 11264
_EPAD = 32 * _TILE_E
_EROW = 128
_ROWS_T = _TILE_E // _EROW
_CH = 8
_NCH = _ROWS_T // _CH
_NC = 2
_NS = 16
_ZROWS = _NU // _NS
_B = 1024
_IDS = 2 * _B

_HIGH = lax.Precision.HIGHEST


def _proj_body(x_ref, ws_ref, wd_ref, as_ref, ad_ref, hs_ref, sc_ref):
    x = x_ref[...]
    hs = jnp.dot(x, ws_ref[...], preferred_element_type=jnp.float32,
                 precision=_HIGH)
    ss = lax.dot_general(as_ref[...], hs, (((1,), (1,)), ((), ())),
                         preferred_element_type=jnp.float32, precision=_HIGH)
    wdv = lax.dot_general(ad_ref[...], wd_ref[...], (((1,), (1,)), ((), ())),
                          preferred_element_type=jnp.float32, precision=_HIGH)
    sd = lax.dot_general(wdv, x, (((1,), (1,)), ((), ())),
                         preferred_element_type=jnp.float32, precision=_HIGH)
    hs_ref[:, 0:_H] = hs
    col = lax.broadcasted_iota(jnp.int32, (x.shape[0], _HW - _H), 1)
    hs_ref[:, _H:_HW] = jnp.where(col == 0, 1.0, 0.0).astype(jnp.float32)
    sc_ref[0:1, :] = ss
    sc_ref[1:2, :] = sd
    sc_ref[2:8, :] = jnp.zeros((6, ss.shape[1]), jnp.float32)


def _proj(x_pad, Ws, Wd, a_s, a_d):
    blk = 1280
    grid = _NPX // blk
    return pl.pallas_call(
        _proj_body,
        grid=(grid,),
        in_specs=[
            pl.BlockSpec((blk, _H), lambda g: (g, 0)),
            pl.BlockSpec((_H, _H), lambda g: (0, 0)),
            pl.BlockSpec((_H, _H), lambda g: (0, 0)),
            pl.BlockSpec((1, _H), lambda g: (0, 0)),
            pl.BlockSpec((1, _H), lambda g: (0, 0)),
        ],
        out_specs=[
            pl.BlockSpec((blk, _HW), lambda g: (g, 0)),
            pl.BlockSpec((8, blk), lambda g: (0, g)),
        ],
        out_shape=[
            jax.ShapeDtypeStruct((_NPX, _HW), jnp.float32),
            jax.ShapeDtypeStruct((8, _NPX), jnp.float32),
        ],
    )(x_pad, Ws, Wd, a_s, a_d)


def _sc_edge_body(src_hbm, dst_hbm, ss_hbm, sd_hbm,
                  ex_hbm, mout_hbm,
                  src_v, dst_v, e_v, ss_v, sd_v, mbuf_v, mall_v, msh_sh):
    cid = lax.axis_index("c")
    sid = lax.axis_index("s")
    tile = cid * _NS + sid
    base = tile * _ROWS_T

    pltpu.sync_copy(src_hbm.at[pl.ds(base, _ROWS_T)], src_v)
    pltpu.sync_copy(dst_hbm.at[pl.ds(base, _ROWS_T)], dst_v)
    pltpu.sync_copy(ss_hbm, ss_v)
    pltpu.sync_copy(sd_hbm, sd_v)

    def passa(b, mx):
        for c in range(_EROW // 16):
            off = c * 16
            isrc = src_v[b, pl.ds(off, 16)]
            idst = dst_v[b, pl.ds(off, 16)]
            e = plsc.load_gather(ss_v, [isrc]) + plsc.load_gather(sd_v, [idst])
            e = jnp.where(e >= 0.0, e, 0.2 * e)
            e_v[b, pl.ds(off, 16)] = e
            mx = jnp.maximum(mx, e)
        return mx
    mx = lax.fori_loop(0, _ROWS_T, passa,
                       jnp.full((16,), -jnp.inf, jnp.float32))

    mbuf_v[...] = mx
    pltpu.sync_copy(mbuf_v, msh_sh.at[pl.ds(sid * 16, 16)])
    plsc.subcore_barrier()
    pltpu.sync_copy(msh_sh, mall_v)
    for t in range(_NS):
        mx = jnp.maximum(mx, mall_v[pl.ds(t * 16, 16)])
    m_core = jnp.max(mx)

    @pl.when(sid == 0)
    def _():
        mbuf_v[...] = jnp.full((16,), m_core, jnp.float32)
        pltpu.sync_copy(mbuf_v, mout_hbm.at[pl.ds(cid * 16, 16)])

    def passb(b, carry):
        for c in range(_EROW // 16):
            off = c * 16
            e = e_v[b, pl.ds(off, 16)]
            e_v[b, pl.ds(off, 16)] = jnp.exp(e - m_core)
        return carry
    lax.fori_loop(0, _ROWS_T, passb, 0)
    pltpu.sync_copy(e_v, ex_hbm.at[pl.ds(base, _ROWS_T)])


def _sc_edge(src2, dst2, ss, sd):
    mesh = plsc.VectorSubcoreMesh(core_axis_name="c", subcore_axis_name="s",
                                  num_cores=_NC, num_subcores=_NS)
    f = pl.kernel(
        _sc_edge_body,
        out_type=[
            jax.ShapeDtypeStruct((_EPAD // _EROW, _EROW), jnp.float32),
            jax.ShapeDtypeStruct((_NC * 16,), jnp.float32),
        ],
        mesh=mesh,
        scratch_types=[
            pltpu.VMEM((_ROWS_T, _EROW), jnp.int32),
            pltpu.VMEM((_ROWS_T, _EROW), jnp.int32),
            pltpu.VMEM((_ROWS_T, _EROW), jnp.float32),
            pltpu.VMEM((_NPX,), jnp.float32),
            pltpu.VMEM((_NPX,), jnp.float32),
            pltpu.VMEM((16,), jnp.float32),
            pltpu.VMEM((_NS * 16,), jnp.float32),
            pltpu.VMEM_SHARED((_NS * 16,), jnp.float32),
        ],
        compiler_params=pltpu.CompilerParams(use_tc_tiling_on_sc=False,
                                             needs_layout_passes=False),
    )
    return f(src2, dst2, ss, sd)


def _sc_msg_body(src_hbm, dst_hbm, ex_hbm, hsaug_hbm, ids_hbm,
                 pout_hbm,
                 src_c, dst_c, ex_c, rows_a, rows_b,
                 u_sh, gsem_a, gsem_b, ssem_a, ssem_b):
    cid = lax.axis_index("c")
    sid = lax.axis_index("s")
    tile = cid * _NS + sid
    base = tile * _ROWS_T
    rows = (rows_a, rows_b)
    gsem = (gsem_a, gsem_b)
    ssem = (ssem_a, ssem_b)

    zv = jnp.zeros((16,), jnp.float32)
    def zrow(r, carry):
        for c in range(_HW // 16):
            rows_a[r, pl.ds(c * 16, 16)] = zv
        return carry
    lax.fori_loop(0, _EROW, zrow, 0)
    nfull = _ZROWS // _EROW
    for k in range(nfull):
        pltpu.sync_copy(rows_a, u_sh.at[pl.ds(sid * _ZROWS + k * _EROW,
                                              _EROW)])
    rem = _ZROWS - nfull * _EROW
    if rem:
        pltpu.sync_copy(rows_a.at[pl.ds(0, rem)],
                        u_sh.at[pl.ds(sid * _ZROWS + nfull * _EROW, rem)])
    plsc.subcore_barrier()

    def scale(buf, r):
        def qloop(q, carry):
            ex16 = ex_c[r, pl.ds(q * 16, 16)]
            for l in range(16):
                rr = q * 16 + l
                ex = ex16[l]
                for c in range(_HW // 16):
                    xv = buf[rr, pl.ds(c * 16, 16)]
                    buf[rr, pl.ds(c * 16, 16)] = xv * ex
            return carry
        lax.fori_loop(0, _EROW // 16, qloop, 0)

    def passc(k, carry):
        pltpu.sync_copy(src_hbm.at[pl.ds(base + k * _CH, _CH)], src_c)
        pltpu.sync_copy(dst_hbm.at[pl.ds(base + k * _CH, _CH)], dst_c)
        pltpu.sync_copy(ex_hbm.at[pl.ds(base + k * _CH, _CH)], ex_c)
        gd = [None, None]
        sd = [None, None]
        gd[0] = pltpu.async_copy(hsaug_hbm.at[src_c.at[0]], rows[0], gsem[0])
        for r in range(_CH):
            p = r % 2
            if r >= 1:
                sd[1 - p].wait()
            if r + 1 < _CH:
                gd[1 - p] = pltpu.async_copy(hsaug_hbm.at[src_c.at[r + 1]],
                                             rows[1 - p], gsem[1 - p])
            gd[p].wait()
            scale(rows[p], r)
            sd[p] = pltpu.async_copy(rows[p], u_sh.at[dst_c.at[r]], ssem[p],
                                     add=True)
        sd[(_CH - 1) % 2].wait()
        return carry
    lax.fori_loop(0, _NCH, passc, 0)
    plsc.subcore_barrier()

    pltpu.sync_copy(ids_hbm.at[pl.ds(sid * _EROW, _EROW)], src_c.at[0])
    pltpu.async_copy(u_sh.at[src_c.at[0]], rows_a, gsem_a).wait()
    pltpu.sync_copy(rows_a, pout_hbm.at[pl.ds(cid * _IDS + sid * _EROW,
                                              _EROW)])


def _sc_msg(src2, dst2, ex2, hsaug, ids1):
    mesh = plsc.VectorSubcoreMesh(core_axis_name="c", subcore_axis_name="s",
                                  num_cores=_NC, num_subcores=_NS)
    f = pl.kernel(
        _sc_msg_body,
        out_type=jax.ShapeDtypeStruct((_NC * _IDS, _HW), jnp.float32),
        mesh=mesh,
        scratch_types=[
            pltpu.VMEM((_CH, _EROW), jnp.int32),
            pltpu.VMEM((_CH, _EROW), jnp.int32),
            pltpu.VMEM((_CH, _EROW), jnp.float32),
            pltpu.VMEM((_EROW, _HW), jnp.float32),
            pltpu.VMEM((_EROW, _HW), jnp.float32),
            pltpu.VMEM_SHARED((_NU, _HW), jnp.float32),
            pltpu.SemaphoreType.DMA,
            pltpu.SemaphoreType.DMA,
            pltpu.SemaphoreType.DMA,
            pltpu.SemaphoreType.DMA,
        ],
        compiler_params=pltpu.CompilerParams(use_tc_tiling_on_sc=False,
                                             needs_layout_passes=False),
    )
    return f(src2, dst2, ex2, hsaug, ids1)


def _cell_body(cf_ref, r1_ref, b1_ref, r2_ref, b2_ref, r3_ref, b3_ref,
               out_ref):
    cf = cf_ref[...]
    n = jnp.sqrt(jnp.sum(cf * cf, axis=1, keepdims=True))
    cf = cf / jnp.maximum(n, 1e-12)
    h = jnp.maximum(jnp.dot(cf, r1_ref[...], preferred_element_type=jnp.float32,
                            precision=_HIGH) + b1_ref[...], 0.0)
    h = jnp.maximum(jnp.dot(h, r2_ref[...], preferred_element_type=jnp.float32,
                            precision=_HIGH) + b2_ref[...], 0.0)
    h = jnp.maximum(jnp.dot(h, r3_ref[...], preferred_element_type=jnp.float32,
                            precision=_HIGH) + b3_ref[...], 0.0)
    out_ref[...] = h


def _cell_mlp(cf, R1, rb1, R2, rb2, R3, rb3):
    return pl.pallas_call(
        _cell_body,
        out_shape=jax.ShapeDtypeStruct((_B, 2 * _H), jnp.float32),
    )(cf, R1, rb1.reshape(1, -1), R2, rb2.reshape(1, -1), R3,
      rb3.reshape(1, -1))


def _final_body(p_ref, m_ref, bdd_ref, cell_ref, s1_ref, sb1_ref, s2_ref,
                sb2_ref, s3_ref, sb3_ref, c_ref, cb_ref, out_ref):
    m = m_ref[...]
    m0 = m[0:1, 0:1]
    m1 = m[0:1, 16:17]
    mg = jnp.maximum(m0, m1)
    f0 = jnp.exp(m0 - mg)
    f1 = jnp.exp(m1 - mg)
    p = p_ref[...]
    comb = p[0:_IDS] * f0 + p[_IDS:2 * _IDS] * f1
    den = comb[:, _H:_H + 1]
    d = comb[:, 0:_H] / (den + 1e-16) + bdd_ref[...]
    d = jnp.maximum(d, 0.0)
    hidden = jnp.concatenate([d[0:_B], d[_B:2 * _B], cell_ref[...]], axis=1)
    n = jnp.sqrt(jnp.sum(hidden * hidden, axis=1, keepdims=True))
    hidden = hidden / jnp.maximum(n, 1e-12)
    h = jnp.maximum(jnp.dot(hidden, s1_ref[...],
                            preferred_element_type=jnp.float32,
                            precision=_HIGH) + sb1_ref[...], 0.0)
    h = jnp.maximum(jnp.dot(h, s2_ref[...], preferred_element_type=jnp.float32,
                            precision=_HIGH) + sb2_ref[...], 0.0)
    h = jnp.maximum(jnp.dot(h, s3_ref[...], preferred_element_type=jnp.float32,
                            precision=_HIGH) + sb3_ref[...], 0.0)
    out_ref[...] = jnp.dot(h, c_ref[...], preferred_element_type=jnp.float32,
                           precision=_HIGH) + cb_ref[...]


def _final_mlp(pout, mout, bdd, cell, S1, sb1, S2, sb2, S3, sb3, C, cb):
    return pl.pallas_call(
        _final_body,
        out_shape=jax.ShapeDtypeStruct((_B, 2), jnp.float32),
    )(pout, mout.reshape(1, -1), bdd.reshape(1, -1), cell, S1,
      sb1.reshape(1, -1), S2, sb2.reshape(1, -1), S3, sb3.reshape(1, -1),
      C, cb.reshape(1, -1))


def kernel(x_drug, x_target, cell_features, edge_index_dd, edge_index_dt,
           edge_index_tt, drug1_id, drug2_id, Wdd_s, Wdd_d, add_s, add_d,
           bdd, Wdt_s, Wdt_d, adt_s, adt_d, bdt, Wtt_s, Wtt_d, att_s, att_d,
           btt, R1, rb1, R2, rb2, R3, rb3, S1, sb1, S2, sb2, S3, sb3, C, cb):
    ei = edge_index_dd.astype(jnp.int32)
    loop = jnp.arange(_ND, dtype=jnp.int32)
    src = jnp.concatenate(
        [ei[0], loop, jnp.zeros((_EPAD - _EL,), jnp.int32)])
    dst = jnp.concatenate(
        [ei[1], loop, jnp.full((_EPAD - _EL,), _ND, jnp.int32)])
    src2 = src.reshape(_EPAD // _EROW, _EROW)
    dst2 = dst.reshape(_EPAD // _EROW, _EROW)
    ids1 = jnp.concatenate([drug1_id.astype(jnp.int32),
                            drug2_id.astype(jnp.int32)])
    x_pad = jnp.pad(x_drug, ((0, _NPX - _ND), (0, 0)))

    cell = _cell_mlp(cell_features, R1, rb1, R2, rb2, R3, rb3)
    hsaug, scores = _proj(x_pad, Wdd_s, Wdd_d, add_s.reshape(1, _H),
                          add_d.reshape(1, _H))
    ex2, mout = _sc_edge(src2, dst2, scores[0], scores[1])
    pout = _sc_msg(src2, dst2, ex2, hsaug, ids1)
    return _final_mlp(pout, mout, bdd, cell, S1, sb1, S2, sb2, S3, sb3, C, cb)


# speedup vs baseline: 9.7056x; 1.2846x over previous
import functools

import jax
import jax.numpy as jnp
from jax import lax
from jax.experimental import pallas as pl
from jax.experimental.pallas import tpu as pltpu
from jax.experimental.pallas import tpu_sc as plsc

_ND = 10000
_H = 128
_HW = 144
_NPX = 10240
_NU = 10112
_E = 320000
_EL = _E + _ND
_TILE_E = 11264
_EPAD = 32 * _TILE_E
_EROW = 128
_ROWS_T = _TILE_E // _EROW
_CH = 8
_NCH = _ROWS_T // _CH
_NC = 2
_NS = 16
_ZROWS = _NU // _NS
_B = 1024
_IDS = 2 * _B
_CW = 12288
_CCH = 1024

_HIGH = lax.Precision.DEFAULT


def _proj_body(x_ref, ws_ref, wd_ref, as_ref, ad_ref, hs_ref, sc_ref):
    x = x_ref[...]
    hs = jnp.dot(x, ws_ref[...], preferred_element_type=jnp.float32,
                 precision=_HIGH)
    ss = lax.dot_general(as_ref[...], hs, (((1,), (1,)), ((), ())),
                         preferred_element_type=jnp.float32, precision=_HIGH)
    wdv = lax.dot_general(ad_ref[...], wd_ref[...], (((1,), (1,)), ((), ())),
                          preferred_element_type=jnp.float32, precision=_HIGH)
    sd = lax.dot_general(wdv, x, (((1,), (1,)), ((), ())),
                         preferred_element_type=jnp.float32, precision=_HIGH)
    hs_ref[:, 0:_H] = hs
    col = lax.broadcasted_iota(jnp.int32, (x.shape[0], _HW - _H), 1)
    hs_ref[:, _H:_HW] = jnp.where(col == 0, 1.0, 0.0).astype(jnp.float32)
    sc_ref[0:1, :] = ss
    sc_ref[1:2, :] = sd
    sc_ref[2:8, :] = jnp.zeros((6, ss.shape[1]), jnp.float32)


def _proj(x_pad, Ws, Wd, a_s, a_d):
    blk = 1280
    grid = _NPX // blk
    return pl.pallas_call(
        _proj_body,
        grid=(grid,),
        in_specs=[
            pl.BlockSpec((blk, _H), lambda g: (g, 0)),
            pl.BlockSpec((_H, _H), lambda g: (0, 0)),
            pl.BlockSpec((_H, _H), lambda g: (0, 0)),
            pl.BlockSpec((1, _H), lambda g: (0, 0)),
            pl.BlockSpec((1, _H), lambda g: (0, 0)),
        ],
        out_specs=[
            pl.BlockSpec((blk, _HW), lambda g: (g, 0)),
            pl.BlockSpec((8, blk), lambda g: (0, g)),
        ],
        out_shape=[
            jax.ShapeDtypeStruct((_NPX, _HW), jnp.float32),
            jax.ShapeDtypeStruct((8, _NPX), jnp.float32),
        ],
    )(x_pad, Ws, Wd, a_s, a_d)


def _sc_edge_body(src_hbm, dst_hbm, ss_hbm, sd_hbm, ids_hbm,
                  csrc_hbm, cdst_hbm, cex_hbm, cnt_hbm, mout_hbm,
                  src_v, dst_v, e_v, ss_v, sd_v, mask_v, ids_v,
                  csrc_v, cdst_v, cex_v, mbuf_v, mall_v, msh_sh):
    cid = lax.axis_index("c")
    sid = lax.axis_index("s")
    tile = cid * _NS + sid
    base = tile * _ROWS_T

    pltpu.sync_copy(src_hbm.at[pl.ds(base, _ROWS_T)], src_v)
    pltpu.sync_copy(dst_hbm.at[pl.ds(base, _ROWS_T)], dst_v)
    pltpu.sync_copy(ss_hbm, ss_v)
    pltpu.sync_copy(sd_hbm, sd_v)
    pltpu.sync_copy(ids_hbm, ids_v)

    zv = jnp.zeros((16,), jnp.float32)
    def zmask(i, carry):
        mask_v[pl.ds(i * 16, 16)] = zv
        return carry
    lax.fori_loop(0, _NPX // 16, zmask, 0)
    ones = jnp.full((16,), 1.0, jnp.float32)
    def smask(j, carry):
        idv = ids_v[pl.ds(j * 16, 16)]
        plsc.store_scatter(mask_v, [idv], ones)
        return carry
    lax.fori_loop(0, _IDS // 16, smask, 0)

    def passa(b, mx):
        for c in range(_EROW // 16):
            off = c * 16
            isrc = src_v[b, pl.ds(off, 16)]
            idst = dst_v[b, pl.ds(off, 16)]
            e = plsc.load_gather(ss_v, [isrc]) + plsc.load_gather(sd_v, [idst])
            e = jnp.where(e >= 0.0, e, 0.2 * e)
            e_v[b, pl.ds(off, 16)] = e
            mx = jnp.maximum(mx, e)
        return mx
    mx = lax.fori_loop(0, _ROWS_T, passa,
                       jnp.full((16,), -jnp.inf, jnp.float32))

    mbuf_v[...] = mx
    pltpu.sync_copy(mbuf_v, msh_sh.at[pl.ds(sid * 16, 16)])
    plsc.subcore_barrier()
    pltpu.sync_copy(msh_sh, mall_v)
    for t in range(_NS):
        mx = jnp.maximum(mx, mall_v[pl.ds(t * 16, 16)])
    m_core = jnp.max(mx)

    @pl.when(sid == 0)
    def _():
        mbuf_v[...] = jnp.full((16,), m_core, jnp.float32)
        pltpu.sync_copy(mbuf_v, mout_hbm.at[pl.ds(cid * 16, 16)])

    def passb(b, off):
        for c in range(_EROW // 16):
            o16 = c * 16
            isrc = src_v[b, pl.ds(o16, 16)]
            idst = dst_v[b, pl.ds(o16, 16)]
            keep = plsc.load_gather(mask_v, [idst]) > 0.5
            ex = jnp.exp(e_v[b, pl.ds(o16, 16)] - m_core)
            plsc.store_compressed(csrc_v.at[pl.ds(off, 16)], isrc, mask=keep)
            plsc.store_compressed(cdst_v.at[pl.ds(off, 16)], idst, mask=keep)
            plsc.store_compressed(cex_v.at[pl.ds(off, 16)], ex, mask=keep)
            pc = plsc.all_reduce_population_count(keep)
            off = off + pc[0]
        return off
    cnt = lax.fori_loop(0, _ROWS_T, passb, jnp.int32(0))

    pad_src = jnp.zeros((16,), jnp.int32)
    pad_dst = jnp.full((16,), _ND, jnp.int32)
    pad_ex = jnp.zeros((16,), jnp.float32)
    for j in range(_CCH // 16):
        csrc_v[pl.ds(cnt + j * 16, 16)] = pad_src
        cdst_v[pl.ds(cnt + j * 16, 16)] = pad_dst
        cex_v[pl.ds(cnt + j * 16, 16)] = pad_ex
    nch = lax.shift_right_logical(cnt + _CCH - 1, 10)

    cbase = tile * _CW
    def wout(k, carry):
        o = k * _CCH
        pltpu.sync_copy(csrc_v.at[pl.ds(o, _CCH)],
                        csrc_hbm.at[pl.ds(cbase + o, _CCH)])
        pltpu.sync_copy(cdst_v.at[pl.ds(o, _CCH)],
                        cdst_hbm.at[pl.ds(cbase + o, _CCH)])
        pltpu.sync_copy(cex_v.at[pl.ds(o, _CCH)],
                        cex_hbm.at[pl.ds(cbase + o, _CCH)])
        return carry
    lax.fori_loop(0, nch, wout, 0)
    mbuf_v[...] = jnp.full((16,), 1.0, jnp.float32) * nch.astype(jnp.float32)
    pltpu.sync_copy(mbuf_v, cnt_hbm.at[pl.ds(tile * 16, 16)])


def _sc_edge(src2, dst2, ss, sd, ids1):
    mesh = plsc.VectorSubcoreMesh(core_axis_name="c", subcore_axis_name="s",
                                  num_cores=_NC, num_subcores=_NS)
    f = pl.kernel(
        _sc_edge_body,
        out_type=[
            jax.ShapeDtypeStruct((32 * _CW,), jnp.int32),
            jax.ShapeDtypeStruct((32 * _CW,), jnp.int32),
            jax.ShapeDtypeStruct((32 * _CW,), jnp.float32),
            jax.ShapeDtypeStruct((32 * 16,), jnp.float32),
            jax.ShapeDtypeStruct((_NC * 16,), jnp.float32),
        ],
        mesh=mesh,
        scratch_types=[
            pltpu.VMEM((_ROWS_T, _EROW), jnp.int32),
            pltpu.VMEM((_ROWS_T, _EROW), jnp.int32),
            pltpu.VMEM((_ROWS_T, _EROW), jnp.float32),
            pltpu.VMEM((_NPX,), jnp.float32),
            pltpu.VMEM((_NPX,), jnp.float32),
            pltpu.VMEM((_NPX,), jnp.float32),
            pltpu.VMEM((_IDS,), jnp.int32),
            pltpu.VMEM((_CW,), jnp.int32),
            pltpu.VMEM((_CW,), jnp.int32),
            pltpu.VMEM((_CW,), jnp.float32),
            pltpu.VMEM((16,), jnp.float32),
            pltpu.VMEM((_NS * 16,), jnp.float32),
            pltpu.VMEM_SHARED((_NS * 16,), jnp.float32),
        ],
        compiler_params=pltpu.CompilerParams(use_tc_tiling_on_sc=False,
                                             needs_layout_passes=False),
    )
    return f(src2, dst2, ss, sd, ids1)


def _sc_msg_body(csrc_hbm, cdst_hbm, cex_hbm, cnt_hbm, hsaug_hbm, ids_hbm,
                 pout_hbm,
                 src_c, dst_c, ex_c, cbuf_v, rows_a, rows_b,
                 u_sh, gsem_a, gsem_b, ssem_a, ssem_b):
    cid = lax.axis_index("c")
    sid = lax.axis_index("s")
    tile = cid * _NS + sid
    cbase = tile * _CW
    rows = (rows_a, rows_b)
    gsem = (gsem_a, gsem_b)
    ssem = (ssem_a, ssem_b)

    zv = jnp.zeros((16,), jnp.float32)
    def zrow(r, carry):
        for c in range(_HW // 16):
            rows_a[r, pl.ds(c * 16, 16)] = zv
        return carry
    lax.fori_loop(0, _EROW, zrow, 0)
    nfull = _ZROWS // _EROW
    for k in range(nfull):
        pltpu.sync_copy(rows_a, u_sh.at[pl.ds(sid * _ZROWS + k * _EROW,
                                              _EROW)])
    rem = _ZROWS - nfull * _EROW
    if rem:
        pltpu.sync_copy(rows_a.at[pl.ds(0, rem)],
                        u_sh.at[pl.ds(sid * _ZROWS + nfull * _EROW, rem)])
    plsc.subcore_barrier()

    pltpu.sync_copy(cnt_hbm.at[pl.ds(tile * 16, 16)], cbuf_v)
    nch = cbuf_v[...][0].astype(jnp.int32)

    def scale(buf, r):
        def qloop(q, carry):
            ex16 = ex_c[pl.ds(r * _EROW + q * 16, 16)]
            for l in range(16):
                rr = q * 16 + l
                ex = ex16[l]
                for c in range(_HW // 16):
                    xv = buf[rr, pl.ds(c * 16, 16)]
                    buf[rr, pl.ds(c * 16, 16)] = xv * ex
            return carry
        lax.fori_loop(0, _EROW // 16, qloop, 0)

    def passc(k, carry):
        pltpu.sync_copy(csrc_hbm.at[pl.ds(cbase + k * _CCH, _CCH)], src_c)
        pltpu.sync_copy(cex_hbm.at[pl.ds(cbase + k * _CCH, _CCH)], ex_c)
        for r in range(_CH):
            pltpu.sync_copy(cdst_hbm.at[pl.ds(cbase + k * _CCH + r * _EROW,
                                              _EROW)], dst_c.at[r])
        gd = [None, None]
        sd = [None, None]
        gd[0] = pltpu.async_copy(hsaug_hbm.at[src_c.at[pl.ds(0, _EROW)]],
                                 rows[0], gsem[0])
        for r in range(_CH):
            p = r % 2
            if r >= 1:
                sd[1 - p].wait()
            if r + 1 < _CH:
                gd[1 - p] = pltpu.async_copy(
                    hsaug_hbm.at[src_c.at[pl.ds((r + 1) * _EROW, _EROW)]],
                    rows[1 - p], gsem[1 - p])
            gd[p].wait()
            scale(rows[p], r)
            sd[p] = pltpu.async_copy(rows[p], u_sh.at[dst_c.at[r]], ssem[p],
                                     add=True)
        sd[(_CH - 1) % 2].wait()
        return carry
    lax.fori_loop(0, nch, passc, 0)
    plsc.subcore_barrier()

    pltpu.sync_copy(ids_hbm.at[pl.ds(sid * _EROW, _EROW)], dst_c.at[0])
    pltpu.async_copy(u_sh.at[dst_c.at[0]], rows_a, gsem_a).wait()
    pltpu.sync_copy(rows_a, pout_hbm.at[pl.ds(cid * _IDS + sid * _EROW,
                                              _EROW)])


def _sc_msg(csrc, cdst, cex, cnts, hsaug, ids1):
    mesh = plsc.VectorSubcoreMesh(core_axis_name="c", subcore_axis_name="s",
                                  num_cores=_NC, num_subcores=_NS)
    f = pl.kernel(
        _sc_msg_body,
        out_type=jax.ShapeDtypeStruct((_NC * _IDS, _HW), jnp.float32),
        mesh=mesh,
        scratch_types=[
            pltpu.VMEM((_CCH,), jnp.int32),
            pltpu.VMEM((_CH, _EROW), jnp.int32),
            pltpu.VMEM((_CCH,), jnp.float32),
            pltpu.VMEM((16,), jnp.float32),
            pltpu.VMEM((_EROW, _HW), jnp.float32),
            pltpu.VMEM((_EROW, _HW), jnp.float32),
            pltpu.VMEM_SHARED((_NU, _HW), jnp.float32),
            pltpu.SemaphoreType.DMA,
            pltpu.SemaphoreType.DMA,
            pltpu.SemaphoreType.DMA,
            pltpu.SemaphoreType.DMA,
        ],
        compiler_params=pltpu.CompilerParams(use_tc_tiling_on_sc=False,
                                             needs_layout_passes=False),
    )
    return f(csrc, cdst, cex, cnts, hsaug, ids1)


def _cell_body(cf_ref, r1_ref, b1_ref, r2_ref, b2_ref, r3_ref, b3_ref,
               out_ref):
    cf = cf_ref[...]
    n = jnp.sqrt(jnp.sum(cf * cf, axis=1, keepdims=True))
    cf = cf / jnp.maximum(n, 1e-12)
    h = jnp.maximum(jnp.dot(cf, r1_ref[...], preferred_element_type=jnp.float32,
                            precision=_HIGH) + b1_ref[...], 0.0)
    h = jnp.maximum(jnp.dot(h, r2_ref[...], preferred_element_type=jnp.float32,
                            precision=_HIGH) + b2_ref[...], 0.0)
    h = jnp.maximum(jnp.dot(h, r3_ref[...], preferred_element_type=jnp.float32,
                            precision=_HIGH) + b3_ref[...], 0.0)
    out_ref[...] = h


def _cell_mlp(cf, R1, rb1, R2, rb2, R3, rb3):
    return pl.pallas_call(
        _cell_body,
        out_shape=jax.ShapeDtypeStruct((_B, 2 * _H), jnp.float32),
    )(cf, R1, rb1.reshape(1, -1), R2, rb2.reshape(1, -1), R3,
      rb3.reshape(1, -1))


def _final_body(p_ref, m_ref, bdd_ref, cell_ref, s1_ref, sb1_ref, s2_ref,
                sb2_ref, s3_ref, sb3_ref, c_ref, cb_ref, out_ref):
    m = m_ref[...]
    m0 = m[0:1, 0:1]
    m1 = m[0:1, 16:17]
    mg = jnp.maximum(m0, m1)
    f0 = jnp.exp(m0 - mg)
    f1 = jnp.exp(m1 - mg)
    p = p_ref[...]
    comb = p[0:_IDS] * f0 + p[_IDS:2 * _IDS] * f1
    den = comb[:, _H:_H + 1]
    d = comb[:, 0:_H] / (den + 1e-16) + bdd_ref[...]
    d = jnp.maximum(d, 0.0)
    hidden = jnp.concatenate([d[0:_B], d[_B:2 * _B], cell_ref[...]], axis=1)
    n = jnp.sqrt(jnp.sum(hidden * hidden, axis=1, keepdims=True))
    hidden = hidden / jnp.maximum(n, 1e-12)
    h = jnp.maximum(jnp.dot(hidden, s1_ref[...],
                            preferred_element_type=jnp.float32,
                            precision=_HIGH) + sb1_ref[...], 0.0)
    h = jnp.maximum(jnp.dot(h, s2_ref[...], preferred_element_type=jnp.float32,
                            precision=_HIGH) + sb2_ref[...], 0.0)
    h = jnp.maximum(jnp.dot(h, s3_ref[...], preferred_element_type=jnp.float32,
                            precision=_HIGH) + sb3_ref[...], 0.0)
    out_ref[...] = jnp.dot(h, c_ref[...], preferred_element_type=jnp.float32,
                           precision=_HIGH) + cb_ref[...]


def _final_mlp(pout, mout, bdd, cell, S1, sb1, S2, sb2, S3, sb3, C, cb):
    return pl.pallas_call(
        _final_body,
        out_shape=jax.ShapeDtypeStruct((_B, 2), jnp.float32),
    )(pout, mout.reshape(1, -1), bdd.reshape(1, -1), cell, S1,
      sb1.reshape(1, -1), S2, sb2.reshape(1, -1), S3, sb3.reshape(1, -1),
      C, cb.reshape(1, -1))


def kernel(x_drug, x_target, cell_features, edge_index_dd, edge_index_dt,
           edge_index_tt, drug1_id, drug2_id, Wdd_s, Wdd_d, add_s, add_d,
           bdd, Wdt_s, Wdt_d, adt_s, adt_d, bdt, Wtt_s, Wtt_d, att_s, att_d,
           btt, R1, rb1, R2, rb2, R3, rb3, S1, sb1, S2, sb2, S3, sb3, C, cb):
    ei = edge_index_dd.astype(jnp.int32)
    loop = jnp.arange(_ND, dtype=jnp.int32)
    src = jnp.concatenate(
        [ei[0], loop, jnp.zeros((_EPAD - _EL,), jnp.int32)])
    dst = jnp.concatenate(
        [ei[1], loop, jnp.full((_EPAD - _EL,), _ND, jnp.int32)])
    src2 = src.reshape(_EPAD // _EROW, _EROW)
    dst2 = dst.reshape(_EPAD // _EROW, _EROW)
    ids1 = jnp.concatenate([drug1_id.astype(jnp.int32),
                            drug2_id.astype(jnp.int32)])
    x_pad = jnp.pad(x_drug, ((0, _NPX - _ND), (0, 0)))

    cell = _cell_mlp(cell_features, R1, rb1, R2, rb2, R3, rb3)
    hsaug, scores = _proj(x_pad, Wdd_s, Wdd_d, add_s.reshape(1, _H),
                          add_d.reshape(1, _H))
    csrc, cdst, cex, cnts, mout = _sc_edge(src2, dst2, scores[0],
                                            scores[1], ids1)
    pout = _sc_msg(csrc, cdst, cex, cnts, hsaug, ids1)
    return _final_mlp(pout, mout, bdd, cell, S1, sb1, S2, sb2, S3, sb3, C, cb)
